# baseline (speedup 1.0000x reference)
import functools
import jax
import jax.numpy as jnp
from jax import lax
from jax.experimental import pallas as pl
from jax.experimental.pallas import tpu as pltpu
from jax.experimental.pallas import tpu_sc as plsc

_N = 10000
_E = 320000
_NSVC = 16384
_ES = 262144
_B = 256
_OUT = 64
_H = 128

_NCORE = 2
_NSUB = 16
_NW = _NCORE * _NSUB
_BLK = 128
_D = 128
_EPAD = 327680
_NACC = 10112

_SC_PARAMS = None
_f32 = jnp.float32
_i32 = jnp.int32
_SC_PARAMS = pltpu.CompilerParams(needs_layout_passes=False,
                                  use_tc_tiling_on_sc=False)


def _iota16():
    return lax.iota(_i32, 16)


def _zeros16():
    return jnp.zeros((16,), _i32)


def _sc_segsum_unweighted(nblocks, nacc, d):
    bpt = nblocks // _NW
    nsup = bpt // 8
    npair = nsup // 2
    rpt = nacc // _NSUB
    mesh = plsc.VectorSubcoreMesh(core_axis_name="c", subcore_axis_name="s")

    def body(src, row3d, col3d, zrows, out, acc,
             bufA, semA, bufB, semB, rP, cP, rQ, cQ):
        c = lax.axis_index("c")
        s = lax.axis_index("s")
        tid = c * _NSUB + s
        base = s * rpt
        sup0 = tid * nsup

        pltpu.sync_copy(zrows.at[pl.ds(base, rpt)], acc.at[pl.ds(base, rpt)])
        plsc.subcore_barrier()

        pltpu.sync_copy(row3d.at[sup0], rP)
        pltpu.sync_copy(col3d.at[sup0], cP)
        pltpu.sync_copy(row3d.at[sup0 + 1], rQ)
        pltpu.sync_copy(col3d.at[sup0 + 1], cQ)

        def outer(k, carry):
            sA = sup0 + 2 * k
            pltpu.async_copy(src.at[rP.at[0]], bufA, semA)
            for j in range(16):
                ridx, cidx = (rP, cP) if j < 8 else (rQ, cQ)
                rj = j % 8
                buf, sem = (bufA, semA) if j % 2 == 0 else (bufB, semB)
                nbuf, nsem = (bufB, semB) if j % 2 == 0 else (bufA, semA)
                if j < 15:
                    nridx = rP if (j + 1) < 8 else rQ
                    pltpu.async_copy(src.at[nridx.at[(j + 1) % 8]],
                                     nbuf, nsem)
                if j == 8:
                    @pl.when(k + 1 < npair)
                    def _():
                        pltpu.sync_copy(row3d.at[sA + 2], rP)
                        pltpu.sync_copy(col3d.at[sA + 2], cP)
                pltpu.make_async_copy(src.at[ridx.at[rj]], buf, sem).wait()
                pltpu.sync_copy(buf, acc.at[cidx.at[rj]], add=True)
                if j == 15:
                    @pl.when(k + 1 < npair)
                    def _():
                        pltpu.sync_copy(row3d.at[sA + 3], rQ)
                        pltpu.sync_copy(col3d.at[sA + 3], cQ)
            return carry

        lax.fori_loop(0, npair, outer, 0)
        plsc.subcore_barrier()
        pltpu.sync_copy(acc.at[pl.ds(base, rpt)], out.at[c, pl.ds(base, rpt)])

    return pl.kernel(
        body,
        out_type=jax.ShapeDtypeStruct((_NCORE, nacc, d), _f32),
        mesh=mesh,
        compiler_params=_SC_PARAMS,
        scratch_types=[
            pltpu.VMEM_SHARED((nacc, d), _f32),
            pltpu.VMEM((_BLK, d), _f32),
            pltpu.SemaphoreType.DMA,
            pltpu.VMEM((_BLK, d), _f32),
            pltpu.SemaphoreType.DMA,
            pltpu.VMEM((8, _BLK), _i32),
            pltpu.VMEM((8, _BLK), _i32),
            pltpu.VMEM((8, _BLK), _i32),
            pltpu.VMEM((8, _BLK), _i32),
        ],
    )


def _sc_segsum_weighted(nblocks, d):
    bpt = nblocks // _NW
    rpt = _NSVC // _NSUB
    mesh = plsc.VectorSubcoreMesh(core_axis_name="c", subcore_axis_name="s")

    def body(src, row2d, col2d, ew2d, zrows, out, acc,
             ridxA, cidxA, bufA, wvA, semA, ridxB, cidxB, bufB, wvB, semB):
        c = lax.axis_index("c")
        s = lax.axis_index("s")
        tid = c * _NSUB + s
        base = s * rpt
        first = tid * bpt

        pltpu.sync_copy(zrows.at[pl.ds(base, rpt)], acc.at[pl.ds(base, rpt)])
        plsc.subcore_barrier()

        pltpu.sync_copy(row2d.at[first], ridxA)
        pltpu.sync_copy(col2d.at[first], cidxA)
        pltpu.sync_copy(ew2d.at[first], wvA)
        pltpu.async_copy(src.at[ridxA], bufA, semA)

        def scale(buf, wv):
            @plsc.parallel_loop(0, _BLK // 16, step=1, unroll=2)
            def grp(g):
                for l in range(16):
                    e = g * 16 + l
                    w = plsc.load_gather(wv, [_zeros16() + e])
                    for t in range(d // 16):
                        sl = pl.ds(t * 16, 16)
                        buf[e, sl] = buf[e, sl] * w

        def step(ii, carry):
            iA = first + 2 * ii
            pltpu.sync_copy(row2d.at[iA + 1], ridxB)
            pltpu.sync_copy(col2d.at[iA + 1], cidxB)
            pltpu.sync_copy(ew2d.at[iA + 1], wvB)
            pltpu.async_copy(src.at[ridxB], bufB, semB)

            pltpu.make_async_copy(src.at[ridxA], bufA, semA).wait()
            scale(bufA, wvA)
            pltpu.sync_copy(bufA, acc.at[cidxA], add=True)

            @pl.when(ii + 1 < bpt // 2)
            def _():
                pltpu.sync_copy(row2d.at[iA + 2], ridxA)
                pltpu.sync_copy(col2d.at[iA + 2], cidxA)
                pltpu.sync_copy(ew2d.at[iA + 2], wvA)
                pltpu.async_copy(src.at[ridxA], bufA, semA)

            pltpu.make_async_copy(src.at[ridxB], bufB, semB).wait()
            scale(bufB, wvB)
            pltpu.sync_copy(bufB, acc.at[cidxB], add=True)
            return carry

        lax.fori_loop(0, bpt // 2, step, 0)
        plsc.subcore_barrier()
        pltpu.sync_copy(acc.at[pl.ds(base, rpt)], out.at[c, pl.ds(base, rpt)])

    return pl.kernel(
        body,
        out_type=jax.ShapeDtypeStruct((_NCORE, _NSVC, d), _f32),
        mesh=mesh,
        compiler_params=_SC_PARAMS,
        scratch_types=[
            pltpu.VMEM_SHARED((_NSVC, d), _f32),
            pltpu.VMEM((_BLK,), _i32),
            pltpu.VMEM((_BLK,), _i32),
            pltpu.VMEM((_BLK, d), _f32),
            pltpu.VMEM((_BLK,), _f32),
            pltpu.SemaphoreType.DMA,
            pltpu.VMEM((_BLK,), _i32),
            pltpu.VMEM((_BLK,), _i32),
            pltpu.VMEM((_BLK, d), _f32),
            pltpu.VMEM((_BLK,), _f32),
            pltpu.SemaphoreType.DMA,
        ],
    )


def _sc_deg(nblocks):
    bpt = nblocks // _NW
    mesh = plsc.VectorSubcoreMesh(core_axis_name="c", subcore_axis_name="s")

    def body(col2d, ew2d, out, tacc, cidx, wv):
        c = lax.axis_index("c")
        s = lax.axis_index("s")
        tid = c * _NSUB + s

        def zrow(r, carry):
            tacc[pl.ds(r * 16, 16)] = jnp.zeros((16,), _f32)
            return carry

        lax.fori_loop(0, _NSVC // 16, zrow, 0)

        def step(i, carry):
            bi = tid * bpt + i
            pltpu.sync_copy(col2d.at[bi], cidx)
            pltpu.sync_copy(ew2d.at[bi], wv)

            def grp(g, c2):
                colv = cidx[pl.ds(g * 16, 16)]
                ewv = wv[pl.ds(g * 16, 16)]
                for l in range(16):
                    m = _iota16() == l
                    plsc.addupdate_scatter(tacc, [colv], ewv, mask=m)
                return c2

            lax.fori_loop(0, _BLK // 16, grp, 0)
            return carry

        lax.fori_loop(0, bpt, step, 0)
        pltpu.sync_copy(tacc, out.at[tid])

    return pl.kernel(
        body,
        out_type=jax.ShapeDtypeStruct((_NW, _NSVC), _f32),
        mesh=mesh,
        compiler_params=_SC_PARAMS,
        scratch_types=[
            pltpu.VMEM((_NSVC,), _f32),
            pltpu.VMEM((_BLK,), _i32),
            pltpu.VMEM((_BLK,), _f32),
        ],
    )


@functools.lru_cache(maxsize=None)
def _seg_unweighted(nblocks, nacc, d):
    return _sc_segsum_unweighted(nblocks, nacc, d)


@functools.lru_cache(maxsize=None)
def _seg_weighted(nblocks, d):
    return _sc_segsum_weighted(nblocks, d)


@functools.lru_cache(maxsize=None)
def _seg_deg(nblocks):
    return _sc_deg(nblocks)


def _bn(z, g, b):
    m = jnp.mean(z, axis=0, keepdims=True)
    v = jnp.mean((z - m) * (z - m), axis=0, keepdims=True)
    return (z - m) / jnp.sqrt(v + 1e-5) * g + b


def _k_gin_front(idx_ref, xr_ref, emb_ref, o_ref):
    oh = (idx_ref[...] == lax.broadcasted_iota(_i32, (1, 100), 1)).astype(_f32)
    emb = jnp.dot(oh, emb_ref[...], preferred_element_type=_f32)
    pad = jnp.zeros((_N, 2), _f32)
    o_ref[...] = jnp.concatenate([emb, xr_ref[...], pad], axis=1)


def _k_gin_mlp(h_ref, a_ref, w1_ref, b1_ref, g1_ref, t1_ref,
               w2_ref, b2_ref, g2_ref, t2_ref, ep_ref, o_ref):
    z = ep_ref[0, 0] * h_ref[...] + a_ref[...]
    z = jnp.dot(z, w1_ref[...], preferred_element_type=_f32) + b1_ref[...]
    z = jnp.maximum(_bn(z, g1_ref[...], t1_ref[...]), 0.0)
    z = jnp.dot(z, w2_ref[...], preferred_element_type=_f32) + b2_ref[...]
    o_ref[...] = jnp.maximum(_bn(z, g2_ref[...], t2_ref[...]), 0.0)


def _k_deg_sum(degp_ref, o_ref):
    o_ref[...] = jnp.sum(degp_ref[...], axis=0, keepdims=True)


def _k_svc_emb(sidx_ref, xsr_ref, emb_ref, o_ref):
    oh = (sidx_ref[...] == lax.broadcasted_iota(_i32, (1, 100), 1)).astype(_f32)
    emb = jnp.dot(oh, emb_ref[...], preferred_element_type=_f32)
    o_ref[...] = jnp.concatenate([emb, xsr_ref[...]], axis=1)


def _k_svc_gx(hs0_ref, w_ref, deg_ref, gx_ref, dinv_ref):
    xw = jnp.dot(hs0_ref[...], w_ref[...], preferred_element_type=_f32)
    deg = deg_ref[...] + 1.0
    dinv = jnp.where(deg > 0, 1.0 / jnp.sqrt(jnp.maximum(deg, 1e-12)), 0.0)
    dinv_ref[...] = dinv
    gx_ref[...] = dinv * xw


def _k_psum(p_ref, o_ref):
    o_ref[...] = p_ref[0] + p_ref[1]


def _k_gcn_z(p_ref, gx_ref, dinv_ref, b_ref, o_ref):
    dinv = dinv_ref[...]
    o_ref[...] = dinv * p_ref[...] + dinv * gx_ref[...] + b_ref[...]


def _k_bnrelu(z_ref, gm_ref, bt_ref, o_ref):
    o_ref[...] = jnp.maximum(_bn(z_ref[...], gm_ref[...], bt_ref[...]), 0.0)


def _k_mm_scaled(h_ref, w_ref, dinv_ref, o_ref):
    o_ref[...] = dinv_ref[...] * jnp.dot(h_ref[...], w_ref[...],
                                         preferred_element_type=_f32)


def _k_lin(h_ref, w_ref, b_ref, o_ref):
    o_ref[...] = jnp.dot(h_ref[...], w_ref[...],
                         preferred_element_type=_f32) + b_ref[...]


def _k_head(h_ref, nw_ref, nb_ref, batch_ref, hsr_ref, o_ref):
    h4 = jnp.dot(h_ref[...], nw_ref[...], preferred_element_type=_f32) + nb_ref[...]
    oh = (batch_ref[...] == lax.broadcasted_iota(_i32, (1, _B), 1)).astype(_f32)
    sums = lax.dot_general(oh, h4, (((0,), (0,)), ((), ())),
                           preferred_element_type=_f32)
    ones = jnp.ones((_N, 1), _f32)
    cnt = lax.dot_general(oh, ones, (((0,), (0,)), ((), ())),
                          preferred_element_type=_f32)
    xg = sums / jnp.maximum(cnt, 1.0)
    xs = jnp.mean(hsr_ref[...], axis=0)
    logits = lax.dot_general(xg, xs, (((1,), (1,)), ((), ())),
                             preferred_element_type=_f32)
    o_ref[...] = 1.0 / (1.0 + jnp.exp(-logits))


def _tc(body, out_shape):
    return pl.pallas_call(body, out_shape=out_shape)


def kernel(x, x_service, edge_attr_service, params, edge_index,
           edge_index_service, batch):
    p = params

    idx = x[:, 0].astype(_i32).reshape(_N, 1)
    xr = x[:, 1:7]
    row = edge_index[0].astype(_i32)
    col = edge_index[1].astype(_i32)
    npad = _EPAD - _E
    row3d = jnp.concatenate([row, jnp.zeros((npad,), _i32)]).reshape(-1, 8, _BLK)
    col3d = jnp.concatenate([col, jnp.full((npad,), _N, _i32)]).reshape(-1, 8, _BLK)
    nblk_gin = _EPAD // _BLK

    sidx = x_service[:, 0].astype(_i32).reshape(_NSVC, 1)
    xsr = x_service[:, 1:5]
    srow2d = edge_index_service[0].astype(_i32).reshape(-1, _BLK)
    scol2d = edge_index_service[1].astype(_i32).reshape(-1, _BLK)
    ew2d = edge_attr_service.astype(_f32).reshape(-1, _BLK)
    nblk_svc = _ES // _BLK

    batch2d = batch.astype(_i32).reshape(_N, 1)
    r1 = lambda a: a.reshape(1, -1)

    def seg_gin(srcs):
        d = srcs.shape[1]
        parts = _seg_unweighted(nblk_gin, _NACC, d)(
            srcs, row3d, col3d, jnp.zeros((_NACC, d), _f32))
        return _tc(_k_psum, jax.ShapeDtypeStruct((_NACC, d), _f32))(
            parts)[:_N]

    def seg_svc(srcs):
        d = srcs.shape[1]
        parts = _seg_weighted(nblk_svc, d)(srcs, srow2d, scol2d, ew2d,
                                           jnp.zeros((_NSVC, d), _f32))
        return _tc(_k_psum, jax.ShapeDtypeStruct((_NSVC, d), _f32))(parts)

    def seg_svc_all(gxa):
        ps = [seg_svc(gxa[:, a:b])
              for a, b in ((0, 96), (96, 192), (192, 256))]
        return jnp.concatenate(ps, axis=1)

    lp = p["gin"][0]
    h = _tc(_k_gin_front, jax.ShapeDtypeStruct((_N, 136), _f32))(
        idx, xr, p["node_emb"])
    w1p = {"W1": jnp.concatenate(
        [lp["W1"], jnp.zeros((2, lp["W1"].shape[1]), _f32)], axis=0)}

    for li, lp in enumerate(p["gin"]):
        if li == 0:
            lp = dict(lp, **w1p)
        aggp = seg_gin(h)
        ep = jnp.reshape(1.0 + lp["eps"].astype(_f32), (1, 1))
        h = _tc(_k_gin_mlp, jax.ShapeDtypeStruct((_N, _H), _f32))(
            h, aggp, lp["W1"], r1(lp["b1"]), r1(lp["g1"]), r1(lp["bt1"]),
            lp["W2"], r1(lp["b2"]), r1(lp["g"]), r1(lp["bt"]), ep)

    degp = _seg_deg(nblk_svc)(scol2d, ew2d)
    deg2d = _tc(_k_deg_sum, jax.ShapeDtypeStruct((1, _NSVC), _f32))(degp)
    deg = deg2d.reshape(_NSVC, 1)

    lp = p["gcn"][0]
    hs0 = _tc(_k_svc_emb, jax.ShapeDtypeStruct((_NSVC, 132), _f32))(
        sidx, xsr, p["svc_emb"])
    gx, dinv = _tc(_k_svc_gx,
                   (jax.ShapeDtypeStruct((_NSVC, 2 * _H), _f32),
                    jax.ShapeDtypeStruct((_NSVC, 1), _f32)))(
        hs0, lp["W"], deg)

    sds = jax.ShapeDtypeStruct((_NSVC, 2 * _H), _f32)
    P = seg_svc_all(gx)
    lp2 = p["gcn"][1]
    z1 = _tc(_k_gcn_z, sds)(P, gx, dinv, r1(lp["b"]))
    hs1 = _tc(_k_bnrelu, sds)(z1, r1(lp["g"]), r1(lp["bt"]))
    gx2 = _tc(_k_mm_scaled, sds)(hs1, lp2["W"], dinv)

    P2 = seg_svc_all(gx2)
    z2 = _tc(_k_gcn_z, sds)(P2, gx2, dinv, r1(lp2["b"]))
    hs2 = _tc(_k_bnrelu, sds)(z2, r1(lp2["g"]), r1(lp2["bt"]))
    hs3 = _tc(_k_lin, jax.ShapeDtypeStruct((_NSVC, _H), _f32))(
        hs2, p["svcLin_W"], r1(p["svcLin_b"]))

    out = _tc(_k_head, jax.ShapeDtypeStruct((_B, _OUT), _f32))(
        h, p["nodeLin_W"], r1(p["nodeLin_b"]), batch2d,
        hs3.reshape(_B, _OUT, _H))
    return out


# speedup vs baseline: 1.1475x; 1.1475x over previous
import functools
import jax
import jax.numpy as jnp
from jax import lax
from jax.experimental import pallas as pl
from jax.experimental.pallas import tpu as pltpu
from jax.experimental.pallas import tpu_sc as plsc

_N = 10000
_E = 320000
_NSVC = 16384
_ES = 262144
_B = 256
_OUT = 64
_H = 128

_NCORE = 2
_NSUB = 16
_NW = _NCORE * _NSUB
_BLK = 128
_D = 128
_EPAD = 327680
_NACC = 10112

_SC_PARAMS = None
_f32 = jnp.float32
_i32 = jnp.int32
_SC_PARAMS = pltpu.CompilerParams(needs_layout_passes=False,
                                  use_tc_tiling_on_sc=False)


def _iota16():
    return lax.iota(_i32, 16)


def _zeros16():
    return jnp.zeros((16,), _i32)


def _sc_segsum_unweighted(nblocks, nacc, d):
    bpt = nblocks // _NW
    nsup = bpt // 8
    npair = nsup // 2
    rpt = nacc // _NSUB
    mesh = plsc.VectorSubcoreMesh(core_axis_name="c", subcore_axis_name="s")

    def body(src, row3d, col3d, zrows, out, acc,
             bufA, semA, bufB, semB, rP, cP, rQ, cQ):
        c = lax.axis_index("c")
        s = lax.axis_index("s")
        tid = c * _NSUB + s
        base = s * rpt
        sup0 = tid * nsup

        pltpu.sync_copy(zrows.at[pl.ds(base, rpt)], acc.at[pl.ds(base, rpt)])
        plsc.subcore_barrier()

        pltpu.sync_copy(row3d.at[sup0], rP)
        pltpu.sync_copy(col3d.at[sup0], cP)
        pltpu.sync_copy(row3d.at[sup0 + 1], rQ)
        pltpu.sync_copy(col3d.at[sup0 + 1], cQ)

        def outer(k, carry):
            sA = sup0 + 2 * k
            pltpu.async_copy(src.at[rP.at[0]], bufA, semA)
            for j in range(16):
                ridx, cidx = (rP, cP) if j < 8 else (rQ, cQ)
                rj = j % 8
                buf, sem = (bufA, semA) if j % 2 == 0 else (bufB, semB)
                nbuf, nsem = (bufB, semB) if j % 2 == 0 else (bufA, semA)
                if j < 15:
                    nridx = rP if (j + 1) < 8 else rQ
                    pltpu.async_copy(src.at[nridx.at[(j + 1) % 8]],
                                     nbuf, nsem)
                if j == 8:
                    @pl.when(k + 1 < npair)
                    def _():
                        pltpu.sync_copy(row3d.at[sA + 2], rP)
                        pltpu.sync_copy(col3d.at[sA + 2], cP)
                pltpu.make_async_copy(src.at[ridx.at[rj]], buf, sem).wait()
                pltpu.sync_copy(buf, acc.at[cidx.at[rj]], add=True)
                if j == 15:
                    @pl.when(k + 1 < npair)
                    def _():
                        pltpu.sync_copy(row3d.at[sA + 3], rQ)
                        pltpu.sync_copy(col3d.at[sA + 3], cQ)
            return carry

        lax.fori_loop(0, npair, outer, 0)
        plsc.subcore_barrier()
        pltpu.sync_copy(acc.at[pl.ds(base, rpt)], out.at[c, pl.ds(base, rpt)])

    return pl.kernel(
        body,
        out_type=jax.ShapeDtypeStruct((_NCORE, nacc, d), _f32),
        mesh=mesh,
        compiler_params=_SC_PARAMS,
        scratch_types=[
            pltpu.VMEM_SHARED((nacc, d), _f32),
            pltpu.VMEM((_BLK, d), _f32),
            pltpu.SemaphoreType.DMA,
            pltpu.VMEM((_BLK, d), _f32),
            pltpu.SemaphoreType.DMA,
            pltpu.VMEM((8, _BLK), _i32),
            pltpu.VMEM((8, _BLK), _i32),
            pltpu.VMEM((8, _BLK), _i32),
            pltpu.VMEM((8, _BLK), _i32),
        ],
    )


def _sc_segsum_weighted(nblocks, d):
    bpt = nblocks // _NW
    nsup = bpt // 8
    npair = nsup // 2
    rpt = _NSVC // _NSUB
    mesh = plsc.VectorSubcoreMesh(core_axis_name="c", subcore_axis_name="s")

    def body(src, row3d, col3d, ew3d, zrows, out, acc,
             bufA, semA, bufB, semB, rP, cP, wP, rQ, cQ, wQ):
        c = lax.axis_index("c")
        s = lax.axis_index("s")
        tid = c * _NSUB + s
        base = s * rpt
        sup0 = tid * nsup

        pltpu.sync_copy(zrows.at[pl.ds(base, rpt)], acc.at[pl.ds(base, rpt)])
        plsc.subcore_barrier()

        pltpu.sync_copy(row3d.at[sup0], rP)
        pltpu.sync_copy(col3d.at[sup0], cP)
        pltpu.sync_copy(ew3d.at[sup0], wP)
        pltpu.sync_copy(row3d.at[sup0 + 1], rQ)
        pltpu.sync_copy(col3d.at[sup0 + 1], cQ)
        pltpu.sync_copy(ew3d.at[sup0 + 1], wQ)

        def scale(buf, wv, rj):
            @plsc.parallel_loop(0, _BLK // 16, step=1, unroll=2)
            def grp(g):
                for l in range(16):
                    e = g * 16 + l
                    w = plsc.load_gather(wv, [_zeros16() + (rj * _BLK + e)])
                    for t in range(d // 16):
                        sl = pl.ds(t * 16, 16)
                        buf[e, sl] = buf[e, sl] * w

        def outer(k, carry):
            sA = sup0 + 2 * k
            pltpu.async_copy(src.at[rP.at[0]], bufA, semA)
            for j in range(16):
                ridx, cidx, widx = (rP, cP, wP) if j < 8 else (rQ, cQ, wQ)
                rj = j % 8
                buf, sem = (bufA, semA) if j % 2 == 0 else (bufB, semB)
                nbuf, nsem = (bufB, semB) if j % 2 == 0 else (bufA, semA)
                if j < 15:
                    nridx = rP if (j + 1) < 8 else rQ
                    pltpu.async_copy(src.at[nridx.at[(j + 1) % 8]],
                                     nbuf, nsem)
                if j == 8:
                    @pl.when(k + 1 < npair)
                    def _():
                        pltpu.sync_copy(row3d.at[sA + 2], rP)
                        pltpu.sync_copy(col3d.at[sA + 2], cP)
                pltpu.make_async_copy(src.at[ridx.at[rj]], buf, sem).wait()
                scale(buf, widx, rj)
                pltpu.sync_copy(buf, acc.at[cidx.at[rj]], add=True)
                if j == 15:
                    @pl.when(k + 1 < npair)
                    def _():
                        pltpu.sync_copy(row3d.at[sA + 3], rQ)
                        pltpu.sync_copy(col3d.at[sA + 3], cQ)
                        pltpu.sync_copy(ew3d.at[sA + 2], wP)
                        pltpu.sync_copy(ew3d.at[sA + 3], wQ)
            return carry

        lax.fori_loop(0, npair, outer, 0)
        plsc.subcore_barrier()
        pltpu.sync_copy(acc.at[pl.ds(base, rpt)], out.at[c, pl.ds(base, rpt)])

    return pl.kernel(
        body,
        out_type=jax.ShapeDtypeStruct((_NCORE, _NSVC, d), _f32),
        mesh=mesh,
        compiler_params=_SC_PARAMS,
        scratch_types=[
            pltpu.VMEM_SHARED((_NSVC, d), _f32),
            pltpu.VMEM((_BLK, d), _f32),
            pltpu.SemaphoreType.DMA,
            pltpu.VMEM((_BLK, d), _f32),
            pltpu.SemaphoreType.DMA,
            pltpu.VMEM((8, _BLK), _i32),
            pltpu.VMEM((8, _BLK), _i32),
            pltpu.VMEM((8 * _BLK,), _f32),
            pltpu.VMEM((8, _BLK), _i32),
            pltpu.VMEM((8, _BLK), _i32),
            pltpu.VMEM((8 * _BLK,), _f32),
        ],
    )


def _sc_deg(nblocks):
    bpt = nblocks // _NW
    mesh = plsc.VectorSubcoreMesh(core_axis_name="c", subcore_axis_name="s")

    def body(col2d, ew2d, out, tacc, cidx, wv):
        c = lax.axis_index("c")
        s = lax.axis_index("s")
        tid = c * _NSUB + s

        def zrow(r, carry):
            tacc[pl.ds(r * 16, 16)] = jnp.zeros((16,), _f32)
            return carry

        lax.fori_loop(0, _NSVC // 16, zrow, 0)

        def step(i, carry):
            bi = tid * bpt + i
            pltpu.sync_copy(col2d.at[bi], cidx)
            pltpu.sync_copy(ew2d.at[bi], wv)

            def grp(g, c2):
                colv = cidx[pl.ds(g * 16, 16)]
                ewv = wv[pl.ds(g * 16, 16)]
                for l in range(16):
                    m = _iota16() == l
                    plsc.addupdate_scatter(tacc, [colv], ewv, mask=m)
                return c2

            lax.fori_loop(0, _BLK // 16, grp, 0)
            return carry

        lax.fori_loop(0, bpt, step, 0)
        pltpu.sync_copy(tacc, out.at[tid])

    return pl.kernel(
        body,
        out_type=jax.ShapeDtypeStruct((_NW, _NSVC), _f32),
        mesh=mesh,
        compiler_params=_SC_PARAMS,
        scratch_types=[
            pltpu.VMEM((_NSVC,), _f32),
            pltpu.VMEM((_BLK,), _i32),
            pltpu.VMEM((_BLK,), _f32),
        ],
    )


@functools.lru_cache(maxsize=None)
def _seg_unweighted(nblocks, nacc, d):
    return _sc_segsum_unweighted(nblocks, nacc, d)


@functools.lru_cache(maxsize=None)
def _seg_weighted(nblocks, d):
    return _sc_segsum_weighted(nblocks, d)


@functools.lru_cache(maxsize=None)
def _seg_deg(nblocks):
    return _sc_deg(nblocks)


def _bn(z, g, b):
    m = jnp.mean(z, axis=0, keepdims=True)
    v = jnp.mean((z - m) * (z - m), axis=0, keepdims=True)
    return (z - m) / jnp.sqrt(v + 1e-5) * g + b


def _k_gin_front(idx_ref, xr_ref, emb_ref, o_ref):
    oh = (idx_ref[...] == lax.broadcasted_iota(_i32, (1, 100), 1)).astype(_f32)
    emb = jnp.dot(oh, emb_ref[...], preferred_element_type=_f32)
    pad = jnp.zeros((_N, 2), _f32)
    o_ref[...] = jnp.concatenate([emb, xr_ref[...], pad], axis=1)


def _k_gin_mlp(h_ref, a_ref, w1_ref, b1_ref, g1_ref, t1_ref,
               w2_ref, b2_ref, g2_ref, t2_ref, ep_ref, o_ref):
    z = ep_ref[0, 0] * h_ref[...] + a_ref[...]
    z = jnp.dot(z, w1_ref[...], preferred_element_type=_f32) + b1_ref[...]
    z = jnp.maximum(_bn(z, g1_ref[...], t1_ref[...]), 0.0)
    z = jnp.dot(z, w2_ref[...], preferred_element_type=_f32) + b2_ref[...]
    o_ref[...] = jnp.maximum(_bn(z, g2_ref[...], t2_ref[...]), 0.0)


def _k_deg_sum(degp_ref, o_ref):
    o_ref[...] = jnp.sum(degp_ref[...], axis=0, keepdims=True)


def _k_svc_emb(sidx_ref, xsr_ref, emb_ref, o_ref):
    oh = (sidx_ref[...] == lax.broadcasted_iota(_i32, (1, 100), 1)).astype(_f32)
    emb = jnp.dot(oh, emb_ref[...], preferred_element_type=_f32)
    o_ref[...] = jnp.concatenate([emb, xsr_ref[...]], axis=1)


def _k_svc_gx(hs0_ref, w_ref, deg_ref, gx_ref, dinv_ref):
    xw = jnp.dot(hs0_ref[...], w_ref[...], preferred_element_type=_f32)
    deg = deg_ref[...] + 1.0
    dinv = jnp.where(deg > 0, 1.0 / jnp.sqrt(jnp.maximum(deg, 1e-12)), 0.0)
    dinv_ref[...] = dinv
    gx_ref[...] = dinv * xw


def _k_psum(p_ref, o_ref):
    o_ref[...] = p_ref[0] + p_ref[1]


def _k_gcn_z(p_ref, gx_ref, dinv_ref, b_ref, o_ref):
    dinv = dinv_ref[...]
    o_ref[...] = dinv * p_ref[...] + dinv * gx_ref[...] + b_ref[...]


def _k_bnrelu(z_ref, gm_ref, bt_ref, o_ref):
    o_ref[...] = jnp.maximum(_bn(z_ref[...], gm_ref[...], bt_ref[...]), 0.0)


def _k_mm_scaled(h_ref, w_ref, dinv_ref, o_ref):
    o_ref[...] = dinv_ref[...] * jnp.dot(h_ref[...], w_ref[...],
                                         preferred_element_type=_f32)


def _k_lin(h_ref, w_ref, b_ref, o_ref):
    o_ref[...] = jnp.dot(h_ref[...], w_ref[...],
                         preferred_element_type=_f32) + b_ref[...]


def _k_head(h_ref, nw_ref, nb_ref, batch_ref, hsr_ref, o_ref):
    h4 = jnp.dot(h_ref[...], nw_ref[...], preferred_element_type=_f32) + nb_ref[...]
    oh = (batch_ref[...] == lax.broadcasted_iota(_i32, (1, _B), 1)).astype(_f32)
    sums = lax.dot_general(oh, h4, (((0,), (0,)), ((), ())),
                           preferred_element_type=_f32)
    ones = jnp.ones((_N, 1), _f32)
    cnt = lax.dot_general(oh, ones, (((0,), (0,)), ((), ())),
                          preferred_element_type=_f32)
    xg = sums / jnp.maximum(cnt, 1.0)
    xs = jnp.mean(hsr_ref[...], axis=0)
    logits = lax.dot_general(xg, xs, (((1,), (1,)), ((), ())),
                             preferred_element_type=_f32)
    o_ref[...] = 1.0 / (1.0 + jnp.exp(-logits))


def _tc(body, out_shape):
    return pl.pallas_call(body, out_shape=out_shape)


def kernel(x, x_service, edge_attr_service, params, edge_index,
           edge_index_service, batch):
    p = params

    idx = x[:, 0].astype(_i32).reshape(_N, 1)
    xr = x[:, 1:7]
    row = edge_index[0].astype(_i32)
    col = edge_index[1].astype(_i32)
    npad = _EPAD - _E
    row3d = jnp.concatenate([row, jnp.zeros((npad,), _i32)]).reshape(-1, 8, _BLK)
    col3d = jnp.concatenate([col, jnp.full((npad,), _N, _i32)]).reshape(-1, 8, _BLK)
    nblk_gin = _EPAD // _BLK

    sidx = x_service[:, 0].astype(_i32).reshape(_NSVC, 1)
    xsr = x_service[:, 1:5]
    srow3d = edge_index_service[0].astype(_i32).reshape(-1, 8, _BLK)
    scol3d = edge_index_service[1].astype(_i32).reshape(-1, 8, _BLK)
    sew3d = edge_attr_service.astype(_f32).reshape(-1, 8 * _BLK)
    scol2d = edge_index_service[1].astype(_i32).reshape(-1, _BLK)
    ew2d = edge_attr_service.astype(_f32).reshape(-1, _BLK)
    nblk_svc = _ES // _BLK

    batch2d = batch.astype(_i32).reshape(_N, 1)
    r1 = lambda a: a.reshape(1, -1)

    def seg_gin(srcs):
        d = srcs.shape[1]
        parts = _seg_unweighted(nblk_gin, _NACC, d)(
            srcs, row3d, col3d, jnp.zeros((_NACC, d), _f32))
        return _tc(_k_psum, jax.ShapeDtypeStruct((_NACC, d), _f32))(
            parts)[:_N]

    def seg_svc(srcs):
        d = srcs.shape[1]
        parts = _seg_weighted(nblk_svc, d)(srcs, srow3d, scol3d, sew3d,
                                           jnp.zeros((_NSVC, d), _f32))
        return _tc(_k_psum, jax.ShapeDtypeStruct((_NSVC, d), _f32))(parts)

    def seg_svc_all(gxa):
        ps = [seg_svc(gxa[:, a:b])
              for a, b in ((0, 96), (96, 192), (192, 256))]
        return jnp.concatenate(ps, axis=1)

    lp = p["gin"][0]
    h = _tc(_k_gin_front, jax.ShapeDtypeStruct((_N, 136), _f32))(
        idx, xr, p["node_emb"])
    w1p = {"W1": jnp.concatenate(
        [lp["W1"], jnp.zeros((2, lp["W1"].shape[1]), _f32)], axis=0)}

    for li, lp in enumerate(p["gin"]):
        if li == 0:
            lp = dict(lp, **w1p)
        aggp = seg_gin(h)
        ep = jnp.reshape(1.0 + lp["eps"].astype(_f32), (1, 1))
        h = _tc(_k_gin_mlp, jax.ShapeDtypeStruct((_N, _H), _f32))(
            h, aggp, lp["W1"], r1(lp["b1"]), r1(lp["g1"]), r1(lp["bt1"]),
            lp["W2"], r1(lp["b2"]), r1(lp["g"]), r1(lp["bt"]), ep)

    degp = _seg_deg(nblk_svc)(scol2d, ew2d)
    deg2d = _tc(_k_deg_sum, jax.ShapeDtypeStruct((1, _NSVC), _f32))(degp)
    deg = deg2d.reshape(_NSVC, 1)

    lp = p["gcn"][0]
    hs0 = _tc(_k_svc_emb, jax.ShapeDtypeStruct((_NSVC, 132), _f32))(
        sidx, xsr, p["svc_emb"])
    gx, dinv = _tc(_k_svc_gx,
                   (jax.ShapeDtypeStruct((_NSVC, 2 * _H), _f32),
                    jax.ShapeDtypeStruct((_NSVC, 1), _f32)))(
        hs0, lp["W"], deg)

    sds = jax.ShapeDtypeStruct((_NSVC, 2 * _H), _f32)
    P = seg_svc_all(gx)
    lp2 = p["gcn"][1]
    z1 = _tc(_k_gcn_z, sds)(P, gx, dinv, r1(lp["b"]))
    hs1 = _tc(_k_bnrelu, sds)(z1, r1(lp["g"]), r1(lp["bt"]))
    gx2 = _tc(_k_mm_scaled, sds)(hs1, lp2["W"], dinv)

    P2 = seg_svc_all(gx2)
    z2 = _tc(_k_gcn_z, sds)(P2, gx2, dinv, r1(lp2["b"]))
    hs2 = _tc(_k_bnrelu, sds)(z2, r1(lp2["g"]), r1(lp2["bt"]))
    hs3 = _tc(_k_lin, jax.ShapeDtypeStruct((_NSVC, _H), _f32))(
        hs2, p["svcLin_W"], r1(p["svcLin_b"]))

    out = _tc(_k_head, jax.ShapeDtypeStruct((_B, _OUT), _f32))(
        h, p["nodeLin_W"], r1(p["nodeLin_b"]), batch2d,
        hs3.reshape(_B, _OUT, _H))
    return out


# speedup vs baseline: 1.2102x; 1.0547x over previous
import functools
import jax
import jax.numpy as jnp
from jax import lax
from jax.experimental import pallas as pl
from jax.experimental.pallas import tpu as pltpu
from jax.experimental.pallas import tpu_sc as plsc

_N = 10000
_E = 320000
_NSVC = 16384
_ES = 262144
_B = 256
_OUT = 64
_H = 128

_NCORE = 2
_NSUB = 16
_NW = _NCORE * _NSUB
_BLK = 128
_D = 128
_EPAD = 327680
_NACC = 10112

_SC_PARAMS = None
_f32 = jnp.float32
_i32 = jnp.int32
_SC_PARAMS = pltpu.CompilerParams(needs_layout_passes=False,
                                  use_tc_tiling_on_sc=False)


def _iota16():
    return lax.iota(_i32, 16)


def _zeros16():
    return jnp.zeros((16,), _i32)


def _sc_segsum_unweighted(nblocks, nacc, d):
    bpt = nblocks // _NW
    nsup = bpt // 8
    npair = nsup // 2
    rpt = nacc // _NSUB
    mesh = plsc.VectorSubcoreMesh(core_axis_name="c", subcore_axis_name="s")

    def body(src, row3d, col3d, zrows, out, acc,
             bufA, semA, bufB, semB, rP, cP, rQ, cQ):
        c = lax.axis_index("c")
        s = lax.axis_index("s")
        tid = c * _NSUB + s
        base = s * rpt
        sup0 = tid * nsup

        pltpu.sync_copy(zrows.at[pl.ds(base, rpt)], acc.at[pl.ds(base, rpt)])
        plsc.subcore_barrier()

        pltpu.sync_copy(row3d.at[sup0], rP)
        pltpu.sync_copy(col3d.at[sup0], cP)
        pltpu.sync_copy(row3d.at[sup0 + 1], rQ)
        pltpu.sync_copy(col3d.at[sup0 + 1], cQ)

        def outer(k, carry):
            sA = sup0 + 2 * k
            pltpu.async_copy(src.at[rP.at[0]], bufA, semA)
            for j in range(16):
                ridx, cidx = (rP, cP) if j < 8 else (rQ, cQ)
                rj = j % 8
                buf, sem = (bufA, semA) if j % 2 == 0 else (bufB, semB)
                nbuf, nsem = (bufB, semB) if j % 2 == 0 else (bufA, semA)
                if j < 15:
                    nridx = rP if (j + 1) < 8 else rQ
                    pltpu.async_copy(src.at[nridx.at[(j + 1) % 8]],
                                     nbuf, nsem)
                if j == 8:
                    @pl.when(k + 1 < npair)
                    def _():
                        pltpu.sync_copy(row3d.at[sA + 2], rP)
                        pltpu.sync_copy(col3d.at[sA + 2], cP)
                pltpu.make_async_copy(src.at[ridx.at[rj]], buf, sem).wait()
                pltpu.sync_copy(buf, acc.at[cidx.at[rj]], add=True)
                if j == 15:
                    @pl.when(k + 1 < npair)
                    def _():
                        pltpu.sync_copy(row3d.at[sA + 3], rQ)
                        pltpu.sync_copy(col3d.at[sA + 3], cQ)
            return carry

        lax.fori_loop(0, npair, outer, 0)
        plsc.subcore_barrier()
        pltpu.sync_copy(acc.at[pl.ds(base, rpt)], out.at[c, pl.ds(base, rpt)])

    return pl.kernel(
        body,
        out_type=jax.ShapeDtypeStruct((_NCORE, nacc, d), _f32),
        mesh=mesh,
        compiler_params=_SC_PARAMS,
        scratch_types=[
            pltpu.VMEM_SHARED((nacc, d), _f32),
            pltpu.VMEM((_BLK, d), _f32),
            pltpu.SemaphoreType.DMA,
            pltpu.VMEM((_BLK, d), _f32),
            pltpu.SemaphoreType.DMA,
            pltpu.VMEM((8, _BLK), _i32),
            pltpu.VMEM((8, _BLK), _i32),
            pltpu.VMEM((8, _BLK), _i32),
            pltpu.VMEM((8, _BLK), _i32),
        ],
    )


def _sc_segsum_weighted(nblocks, d):
    bpt = nblocks // _NW
    nsup = bpt // 8
    npair = nsup // 2
    rpt = _NSVC // _NSUB
    mesh = plsc.VectorSubcoreMesh(core_axis_name="c", subcore_axis_name="s")

    def body(src, row3d, col3d, ew3d, zrows, out, acc,
             bufA, semA, bufB, semB, rP, cP, wP, rQ, cQ, wQ):
        c = lax.axis_index("c")
        s = lax.axis_index("s")
        tid = c * _NSUB + s
        base = s * rpt
        sup0 = tid * nsup

        pltpu.sync_copy(zrows.at[pl.ds(base, rpt)], acc.at[pl.ds(base, rpt)])
        plsc.subcore_barrier()

        pltpu.sync_copy(row3d.at[sup0], rP)
        pltpu.sync_copy(col3d.at[sup0], cP)
        pltpu.sync_copy(ew3d.at[sup0], wP)
        pltpu.sync_copy(row3d.at[sup0 + 1], rQ)
        pltpu.sync_copy(col3d.at[sup0 + 1], cQ)
        pltpu.sync_copy(ew3d.at[sup0 + 1], wQ)

        def scale(buf, wv, rj):
            @plsc.parallel_loop(0, _BLK // 16, step=1, unroll=2)
            def grp(g):
                for l in range(16):
                    e = g * 16 + l
                    w = plsc.load_gather(wv, [_zeros16() + (rj * _BLK + e)])
                    for t in range(d // 16):
                        sl = pl.ds(t * 16, 16)
                        buf[e, sl] = buf[e, sl] * w

        def outer(k, carry):
            sA = sup0 + 2 * k
            pltpu.async_copy(src.at[rP.at[0]], bufA, semA)
            for j in range(16):
                ridx, cidx, widx = (rP, cP, wP) if j < 8 else (rQ, cQ, wQ)
                rj = j % 8
                buf, sem = (bufA, semA) if j % 2 == 0 else (bufB, semB)
                nbuf, nsem = (bufB, semB) if j % 2 == 0 else (bufA, semA)
                if j < 15:
                    nridx = rP if (j + 1) < 8 else rQ
                    pltpu.async_copy(src.at[nridx.at[(j + 1) % 8]],
                                     nbuf, nsem)
                if j == 8:
                    @pl.when(k + 1 < npair)
                    def _():
                        pltpu.sync_copy(row3d.at[sA + 2], rP)
                        pltpu.sync_copy(col3d.at[sA + 2], cP)
                pltpu.make_async_copy(src.at[ridx.at[rj]], buf, sem).wait()
                scale(buf, widx, rj)
                pltpu.sync_copy(buf, acc.at[cidx.at[rj]], add=True)
                if j == 15:
                    @pl.when(k + 1 < npair)
                    def _():
                        pltpu.sync_copy(row3d.at[sA + 3], rQ)
                        pltpu.sync_copy(col3d.at[sA + 3], cQ)
                        pltpu.sync_copy(ew3d.at[sA + 2], wP)
                        pltpu.sync_copy(ew3d.at[sA + 3], wQ)
            return carry

        lax.fori_loop(0, npair, outer, 0)
        plsc.subcore_barrier()
        pltpu.sync_copy(acc.at[pl.ds(base, rpt)], out.at[c, pl.ds(base, rpt)])

    return pl.kernel(
        body,
        out_type=jax.ShapeDtypeStruct((_NCORE, _NSVC, d), _f32),
        mesh=mesh,
        compiler_params=_SC_PARAMS,
        scratch_types=[
            pltpu.VMEM_SHARED((_NSVC, d), _f32),
            pltpu.VMEM((_BLK, d), _f32),
            pltpu.SemaphoreType.DMA,
            pltpu.VMEM((_BLK, d), _f32),
            pltpu.SemaphoreType.DMA,
            pltpu.VMEM((8, _BLK), _i32),
            pltpu.VMEM((8, _BLK), _i32),
            pltpu.VMEM((8 * _BLK,), _f32),
            pltpu.VMEM((8, _BLK), _i32),
            pltpu.VMEM((8, _BLK), _i32),
            pltpu.VMEM((8 * _BLK,), _f32),
        ],
    )


def _sc_deg(nblocks):
    bpt = nblocks // _NW
    mesh = plsc.VectorSubcoreMesh(core_axis_name="c", subcore_axis_name="s")

    def body(col2d, ew2d, out, tacc, cidx, wv):
        c = lax.axis_index("c")
        s = lax.axis_index("s")
        tid = c * _NSUB + s

        def zrow(r, carry):
            tacc[pl.ds(r * 16, 16)] = jnp.zeros((16,), _f32)
            return carry

        lax.fori_loop(0, _NSVC // 16, zrow, 0)

        def step(i, carry):
            bi = tid * bpt + i
            pltpu.sync_copy(col2d.at[bi], cidx)
            pltpu.sync_copy(ew2d.at[bi], wv)

            def grp(g, c2):
                colv = cidx[pl.ds(g * 16, 16)]
                ewv = wv[pl.ds(g * 16, 16)]
                for l in range(16):
                    m = _iota16() == l
                    plsc.addupdate_scatter(tacc, [colv], ewv, mask=m)
                return c2

            lax.fori_loop(0, _BLK // 16, grp, 0)
            return carry

        lax.fori_loop(0, bpt, step, 0)
        pltpu.sync_copy(tacc, out.at[tid])

    return pl.kernel(
        body,
        out_type=jax.ShapeDtypeStruct((_NW, _NSVC), _f32),
        mesh=mesh,
        compiler_params=_SC_PARAMS,
        scratch_types=[
            pltpu.VMEM((_NSVC,), _f32),
            pltpu.VMEM((_BLK,), _i32),
            pltpu.VMEM((_BLK,), _f32),
        ],
    )


@functools.lru_cache(maxsize=None)
def _seg_unweighted(nblocks, nacc, d):
    return _sc_segsum_unweighted(nblocks, nacc, d)


@functools.lru_cache(maxsize=None)
def _seg_weighted(nblocks, d):
    return _sc_segsum_weighted(nblocks, d)


@functools.lru_cache(maxsize=None)
def _seg_deg(nblocks):
    return _sc_deg(nblocks)


def _bn(z, g, b):
    m = jnp.mean(z, axis=0, keepdims=True)
    v = jnp.mean((z - m) * (z - m), axis=0, keepdims=True)
    return (z - m) / jnp.sqrt(v + 1e-5) * g + b


def _k_gin_front(idx_ref, xr_ref, emb_ref, o_ref):
    oh = (idx_ref[...] == lax.broadcasted_iota(_i32, (1, 100), 1)).astype(_f32)
    emb = jnp.dot(oh, emb_ref[...], preferred_element_type=_f32)
    pad = jnp.zeros((_N, 2), _f32)
    o_ref[...] = jnp.concatenate([emb, xr_ref[...], pad], axis=1)


def _k_gin_mlp(h_ref, a_ref, w1_ref, b1_ref, g1_ref, t1_ref,
               w2_ref, b2_ref, g2_ref, t2_ref, ep_ref, o_ref):
    z = ep_ref[0, 0] * h_ref[...] + a_ref[...]
    z = jnp.dot(z, w1_ref[...], preferred_element_type=_f32) + b1_ref[...]
    z = jnp.maximum(_bn(z, g1_ref[...], t1_ref[...]), 0.0)
    z = jnp.dot(z, w2_ref[...], preferred_element_type=_f32) + b2_ref[...]
    o_ref[...] = jnp.maximum(_bn(z, g2_ref[...], t2_ref[...]), 0.0)


def _k_deg_sum(degp_ref, o_ref):
    o_ref[...] = jnp.sum(degp_ref[...], axis=0, keepdims=True)


def _k_svc_emb(sidx_ref, xsr_ref, emb_ref, o_ref):
    oh = (sidx_ref[...] == lax.broadcasted_iota(_i32, (1, 100), 1)).astype(_f32)
    emb = jnp.dot(oh, emb_ref[...], preferred_element_type=_f32)
    o_ref[...] = jnp.concatenate([emb, xsr_ref[...]], axis=1)


def _k_svc_g(hs0_ref, deg_ref, g_ref, dinv_ref):
    deg = deg_ref[...] + 1.0
    dinv = jnp.where(deg > 0, 1.0 / jnp.sqrt(jnp.maximum(deg, 1e-12)), 0.0)
    dinv_ref[...] = dinv
    pad = jnp.zeros((_NSVC, 12), _f32)
    g_ref[...] = dinv * jnp.concatenate([hs0_ref[...], pad], axis=1)


def _k_gcn_l1(p_ref, g_ref, dinv_ref, w_ref, b_ref, o_ref):
    dinv = dinv_ref[...]
    pre = dinv * p_ref[...] + dinv * g_ref[...]
    o_ref[...] = jnp.dot(pre, w_ref[...],
                         preferred_element_type=_f32) + b_ref[...]


def _k_psum(p_ref, o_ref):
    o_ref[...] = p_ref[0] + p_ref[1]


def _k_gcn_z(p_ref, gx_ref, dinv_ref, b_ref, o_ref):
    dinv = dinv_ref[...]
    o_ref[...] = dinv * p_ref[...] + dinv * gx_ref[...] + b_ref[...]


def _k_bnrelu(z_ref, gm_ref, bt_ref, o_ref):
    o_ref[...] = jnp.maximum(_bn(z_ref[...], gm_ref[...], bt_ref[...]), 0.0)


def _k_mm_scaled(h_ref, w_ref, dinv_ref, o_ref):
    o_ref[...] = dinv_ref[...] * jnp.dot(h_ref[...], w_ref[...],
                                         preferred_element_type=_f32)


def _k_lin(h_ref, w_ref, b_ref, o_ref):
    o_ref[...] = jnp.dot(h_ref[...], w_ref[...],
                         preferred_element_type=_f32) + b_ref[...]


def _k_head(h_ref, nw_ref, nb_ref, batch_ref, hsr_ref, o_ref):
    h4 = jnp.dot(h_ref[...], nw_ref[...], preferred_element_type=_f32) + nb_ref[...]
    oh = (batch_ref[...] == lax.broadcasted_iota(_i32, (1, _B), 1)).astype(_f32)
    sums = lax.dot_general(oh, h4, (((0,), (0,)), ((), ())),
                           preferred_element_type=_f32)
    ones = jnp.ones((_N, 1), _f32)
    cnt = lax.dot_general(oh, ones, (((0,), (0,)), ((), ())),
                          preferred_element_type=_f32)
    xg = sums / jnp.maximum(cnt, 1.0)
    xs = jnp.mean(hsr_ref[...], axis=0)
    logits = lax.dot_general(xg, xs, (((1,), (1,)), ((), ())),
                             preferred_element_type=_f32)
    o_ref[...] = 1.0 / (1.0 + jnp.exp(-logits))


def _tc(body, out_shape):
    return pl.pallas_call(body, out_shape=out_shape)


def kernel(x, x_service, edge_attr_service, params, edge_index,
           edge_index_service, batch):
    p = params

    idx = x[:, 0].astype(_i32).reshape(_N, 1)
    xr = x[:, 1:7]
    row = edge_index[0].astype(_i32)
    col = edge_index[1].astype(_i32)
    npad = _EPAD - _E
    row3d = jnp.concatenate([row, jnp.zeros((npad,), _i32)]).reshape(-1, 8, _BLK)
    col3d = jnp.concatenate([col, jnp.full((npad,), _N, _i32)]).reshape(-1, 8, _BLK)
    nblk_gin = _EPAD // _BLK

    sidx = x_service[:, 0].astype(_i32).reshape(_NSVC, 1)
    xsr = x_service[:, 1:5]
    srow3d = edge_index_service[0].astype(_i32).reshape(-1, 8, _BLK)
    scol3d = edge_index_service[1].astype(_i32).reshape(-1, 8, _BLK)
    sew3d = edge_attr_service.astype(_f32).reshape(-1, 8 * _BLK)
    scol2d = edge_index_service[1].astype(_i32).reshape(-1, _BLK)
    ew2d = edge_attr_service.astype(_f32).reshape(-1, _BLK)
    nblk_svc = _ES // _BLK

    batch2d = batch.astype(_i32).reshape(_N, 1)
    r1 = lambda a: a.reshape(1, -1)

    def seg_gin(srcs):
        d = srcs.shape[1]
        parts = _seg_unweighted(nblk_gin, _NACC, d)(
            srcs, row3d, col3d, jnp.zeros((_NACC, d), _f32))
        return _tc(_k_psum, jax.ShapeDtypeStruct((_NACC, d), _f32))(
            parts)[:_N]

    def seg_svc(srcs):
        d = srcs.shape[1]
        parts = _seg_weighted(nblk_svc, d)(srcs, srow3d, scol3d, sew3d,
                                           jnp.zeros((_NSVC, d), _f32))
        return _tc(_k_psum, jax.ShapeDtypeStruct((_NSVC, d), _f32))(parts)

    def seg_svc_all(gxa):
        ps = [seg_svc(gxa[:, a:b])
              for a, b in ((0, 96), (96, 192), (192, 256))]
        return jnp.concatenate(ps, axis=1)

    lp = p["gin"][0]
    h = _tc(_k_gin_front, jax.ShapeDtypeStruct((_N, 136), _f32))(
        idx, xr, p["node_emb"])
    w1p = {"W1": jnp.concatenate(
        [lp["W1"], jnp.zeros((2, lp["W1"].shape[1]), _f32)], axis=0)}

    for li, lp in enumerate(p["gin"]):
        if li == 0:
            lp = dict(lp, **w1p)
        aggp = seg_gin(h)
        ep = jnp.reshape(1.0 + lp["eps"].astype(_f32), (1, 1))
        h = _tc(_k_gin_mlp, jax.ShapeDtypeStruct((_N, _H), _f32))(
            h, aggp, lp["W1"], r1(lp["b1"]), r1(lp["g1"]), r1(lp["bt1"]),
            lp["W2"], r1(lp["b2"]), r1(lp["g"]), r1(lp["bt"]), ep)

    degp = _seg_deg(nblk_svc)(scol2d, ew2d)
    deg2d = _tc(_k_deg_sum, jax.ShapeDtypeStruct((1, _NSVC), _f32))(degp)
    deg = deg2d.reshape(_NSVC, 1)

    lp = p["gcn"][0]
    hs0 = _tc(_k_svc_emb, jax.ShapeDtypeStruct((_NSVC, 132), _f32))(
        sidx, xsr, p["svc_emb"])
    g1, dinv = _tc(_k_svc_g,
                   (jax.ShapeDtypeStruct((_NSVC, 144), _f32),
                    jax.ShapeDtypeStruct((_NSVC, 1), _f32)))(hs0, deg)

    sds = jax.ShapeDtypeStruct((_NSVC, 2 * _H), _f32)
    P1 = jnp.concatenate([seg_svc(g1[:, :64]), seg_svc(g1[:, 64:144])],
                         axis=1)
    w1p = jnp.concatenate([lp["W"], jnp.zeros((12, lp["W"].shape[1]), _f32)],
                          axis=0)
    lp2 = p["gcn"][1]
    z1 = _tc(_k_gcn_l1, sds)(P1, g1, dinv, w1p, r1(lp["b"]))
    hs1 = _tc(_k_bnrelu, sds)(z1, r1(lp["g"]), r1(lp["bt"]))
    gx2 = _tc(_k_mm_scaled, sds)(hs1, lp2["W"], dinv)

    P2 = seg_svc_all(gx2)
    z2 = _tc(_k_gcn_z, sds)(P2, gx2, dinv, r1(lp2["b"]))
    hs2 = _tc(_k_bnrelu, sds)(z2, r1(lp2["g"]), r1(lp2["bt"]))
    hs3 = _tc(_k_lin, jax.ShapeDtypeStruct((_NSVC, _H), _f32))(
        hs2, p["svcLin_W"], r1(p["svcLin_b"]))

    out = _tc(_k_head, jax.ShapeDtypeStruct((_B, _OUT), _f32))(
        h, p["nodeLin_W"], r1(p["nodeLin_b"]), batch2d,
        hs3.reshape(_B, _OUT, _H))
    return out
